# baseline (speedup 1.0000x reference)
import jax
import jax.numpy as jnp
from jax.experimental import pallas as pl
from jax.experimental.pallas import tpu as pltpu

_COL_A = 5
_COL_B = 1000
_ROWS = 8192
_COLS = 4096
_TA = (_COL_A // 128) * 128
_TB = (_COL_B // 128) * 128
_OA = _COL_A - _TA
_OB = _COL_B - _TB
_NBIG = 4


def _dma_body(x_hbm, o_hbm, va, vb, sems):
    rows = _ROWS // _NBIG
    big = [
        pltpu.make_async_copy(
            x_hbm.at[pl.ds(k * rows, rows)],
            o_hbm.at[pl.ds(k * rows, rows)],
            sems.at[k],
        )
        for k in range(_NBIG)
    ]
    ta_in = pltpu.make_async_copy(x_hbm.at[:, pl.ds(_TA, 128)], va, sems.at[_NBIG])
    tb_in = pltpu.make_async_copy(x_hbm.at[:, pl.ds(_TB, 128)], vb, sems.at[_NBIG + 1])
    for c in big:
        c.start()
    ta_in.start()
    tb_in.start()
    ta_in.wait()
    tb_in.wait()
    a_col = va[:, _OA:_OA + 1]
    b_col = vb[:, _OB:_OB + 1]
    va[:, _OA:_OA + 1] = b_col
    vb[:, _OB:_OB + 1] = a_col
    for c in big:
        c.wait()
    ta_out = pltpu.make_async_copy(va, o_hbm.at[:, pl.ds(_TA, 128)], sems.at[_NBIG + 2])
    tb_out = pltpu.make_async_copy(vb, o_hbm.at[:, pl.ds(_TB, 128)], sems.at[_NBIG + 3])
    ta_out.start()
    tb_out.start()
    ta_out.wait()
    tb_out.wait()


def kernel(x):
    return pl.pallas_call(
        _dma_body,
        in_specs=[pl.BlockSpec(memory_space=pl.ANY)],
        out_specs=pl.BlockSpec(memory_space=pl.ANY),
        out_shape=jax.ShapeDtypeStruct((_ROWS, _COLS), x.dtype),
        scratch_shapes=[
            pltpu.VMEM((_ROWS, 128), jnp.float32),
            pltpu.VMEM((_ROWS, 128), jnp.float32),
            pltpu.SemaphoreType.DMA((_NBIG + 4,)),
        ],
    )(x)


# speedup vs baseline: 48.1916x; 48.1916x over previous
import jax
import jax.numpy as jnp
from jax.experimental import pallas as pl

_COL_A = 5
_COL_B = 1000
_ROWS = 8192
_COLS = 4096
_BLK = 256


def _swap_body(x_ref, o_ref):
    xv = x_ref[...]
    o_ref[...] = xv
    o_ref[:, _COL_A:_COL_A + 1] = xv[:, _COL_B:_COL_B + 1]
    o_ref[:, _COL_B:_COL_B + 1] = xv[:, _COL_A:_COL_A + 1]


def kernel(x):
    return pl.pallas_call(
        _swap_body,
        grid=(_ROWS // _BLK,),
        in_specs=[pl.BlockSpec((_BLK, _COLS), lambda i: (i, 0))],
        out_specs=pl.BlockSpec((_BLK, _COLS), lambda i: (i, 0)),
        out_shape=jax.ShapeDtypeStruct((_ROWS, _COLS), x.dtype),
    )(x)


# speedup vs baseline: 49.0977x; 1.0188x over previous
import jax
import jax.numpy as jnp
from jax.experimental import pallas as pl

_COL_A = 5
_COL_B = 1000
_ROWS = 8192
_COLS = 4096
_BLK = 512


def _swap_body(x_ref, o_ref):
    xv = x_ref[...]
    o_ref[...] = xv
    o_ref[:, _COL_A:_COL_A + 1] = xv[:, _COL_B:_COL_B + 1]
    o_ref[:, _COL_B:_COL_B + 1] = xv[:, _COL_A:_COL_A + 1]


def kernel(x):
    return pl.pallas_call(
        _swap_body,
        grid=(_ROWS // _BLK,),
        in_specs=[pl.BlockSpec((_BLK, _COLS), lambda i: (i, 0))],
        out_specs=pl.BlockSpec((_BLK, _COLS), lambda i: (i, 0)),
        out_shape=jax.ShapeDtypeStruct((_ROWS, _COLS), x.dtype),
    )(x)
